# Initial kernel scaffold; baseline (speedup 1.0000x reference)
import jax
import jax.numpy as jnp
from jax.experimental import pallas as pl


def kernel(x0, x1, x2, iw0, ib0, lw0, lb0, iw1, ib1, lw1, lb1, iw2, ib2, lw2, lb2):
    raise NotImplementedError("write your pallas kernel here")


# speedup vs baseline: 1.2910x; 1.2910x over previous
import jax
import jax.numpy as jnp
from jax import lax
from jax.experimental import pallas as pl
from jax.experimental.pallas import tpu as pltpu

BF16 = jnp.bfloat16


def _lateral_kernel(x_ref, w_ref, b_ref, o_ref):
    x = x_ref[0].astype(BF16)
    acc = lax.dot_general(x, w_ref[...], (((0,), (0,)), ((), ())),
                          preferred_element_type=jnp.float32)
    o_ref[0] = (acc + b_ref[...]).astype(o_ref.dtype)


def _lateral_td_kernel(x_ref, w_ref, b_ref, td_ref, o_ref):
    x = x_ref[0].astype(BF16)
    acc = lax.dot_general(x, w_ref[...], (((0,), (0,)), ((), ())),
                          preferred_element_type=jnp.float32)
    td = td_ref[0]
    r2, w2, c = td.shape
    td = jnp.broadcast_to(td[:, :, None, :], (r2, w2, 2, c)).reshape(r2, 2 * w2, c)
    td = jnp.broadcast_to(td[:, None, :, :], (r2, 2, 2 * w2, c)).reshape(2 * r2, 2 * w2, c)
    acc = acc + b_ref[...] + td.reshape(-1, c).astype(jnp.float32)
    o_ref[0] = acc.astype(o_ref.dtype)


def _lateral(x_nchw, w, b, rows_per_tile, td=None):
    N, Cin, H, W = x_nchw.shape
    Cout = w.shape[0]
    xm = x_nchw.reshape(N, Cin, H * W)
    wm = jnp.transpose(w.reshape(Cout, Cin)).astype(BF16)
    bm = b.reshape(1, Cout)

    R = rows_per_tile
    TM = R * W
    nt = (H * W) // TM

    in_specs = [
        pl.BlockSpec((1, Cin, TM), lambda n, t: (n, 0, t)),
        pl.BlockSpec((Cin, Cout), lambda n, t: (0, 0)),
        pl.BlockSpec((1, Cout), lambda n, t: (0, 0)),
    ]
    args = [xm, wm, bm]
    kfn = _lateral_kernel
    if td is not None:
        r2 = R // 2
        w2 = W // 2
        in_specs.append(pl.BlockSpec((1, r2, w2, Cout), lambda n, t: (n, t, 0, 0)))
        args.append(td)
        kfn = _lateral_td_kernel

    out = pl.pallas_call(
        kfn,
        out_shape=jax.ShapeDtypeStruct((N, H * W, Cout), BF16),
        grid=(N, nt),
        in_specs=in_specs,
        out_specs=pl.BlockSpec((1, TM, Cout), lambda n, t: (n, t, 0)),
        compiler_params=pltpu.CompilerParams(
            dimension_semantics=("parallel", "parallel"),
            vmem_limit_bytes=100 * 1024 * 1024,
        ),
    )(*args)
    return out.reshape(N, H, W, Cout)


def _smooth_kernel(xa_ref, xb1_ref, xb2_ref, w_ref, b_ref, o_ref,
                   xrow_ref, acc_ref):
    TH = xa_ref.shape[1]
    Wp = xa_ref.shape[2]
    C = xa_ref.shape[3]
    W = Wp - 2
    Cout = o_ref.shape[1]

    xrow_ref[0:TH] = xa_ref[0]
    xrow_ref[TH] = xb1_ref[0, 0]
    xrow_ref[TH + 1] = xb2_ref[0, 0]

    acc_ref[...] = jnp.zeros((TH * W, Cout), jnp.float32)
    for dy in range(3):
        for dx in range(3):
            patch = xrow_ref[dy:dy + TH, dx:dx + W, :]
            acc_ref[...] += jnp.dot(
                patch.reshape(TH * W, C),
                w_ref[dy * 3 + dx],
                preferred_element_type=jnp.float32,
            )

    o_ref[0] = jnp.transpose(acc_ref[...]) + b_ref[...]


def _smooth(x_nhwc, w, b, rows_per_tile):
    N, H, W, C = x_nhwc.shape
    Cout = w.shape[0]
    xp = jnp.pad(x_nhwc, ((0, 0), (1, 1), (1, 1), (0, 0)))
    wk = jnp.transpose(w, (2, 3, 1, 0)).reshape(9, C, Cout).astype(BF16)
    bm = b.reshape(Cout, 1)

    TH = rows_per_tile
    Wp = W + 2
    grid = (N, H // TH)

    out = pl.pallas_call(
        _smooth_kernel,
        out_shape=jax.ShapeDtypeStruct((N, Cout, H * W), jnp.float32),
        grid=grid,
        in_specs=[
            pl.BlockSpec((1, TH, Wp, C), lambda n, t: (n, t, 0, 0)),
            pl.BlockSpec((1, 1, Wp, C), lambda n, t: (n, (t + 1) * TH, 0, 0)),
            pl.BlockSpec((1, 1, Wp, C), lambda n, t: (n, (t + 1) * TH + 1, 0, 0)),
            pl.BlockSpec((9, C, Cout), lambda n, t: (0, 0, 0)),
            pl.BlockSpec((Cout, 1), lambda n, t: (0, 0)),
        ],
        out_specs=pl.BlockSpec((1, Cout, TH * W), lambda n, t: (n, 0, t)),
        scratch_shapes=[
            pltpu.VMEM((TH + 2, Wp, C), BF16),
            pltpu.VMEM((TH * W, Cout), jnp.float32),
        ],
        compiler_params=pltpu.CompilerParams(
            dimension_semantics=("parallel", "parallel"),
            vmem_limit_bytes=100 * 1024 * 1024,
        ),
    )(xp, xp, xp, wk, bm)
    return out.reshape(N, Cout, H, W)


def kernel(x0, x1, x2, iw0, ib0, lw0, lb0, iw1, ib1, lw1, lb1, iw2, ib2, lw2, lb2):
    inner2 = _lateral(x2, iw2, ib2, rows_per_tile=32)
    r2 = _smooth(inner2, lw2, lb2, rows_per_tile=32)

    inner1 = _lateral(x1, iw1, ib1, rows_per_tile=32, td=inner2)
    r1 = _smooth(inner1, lw1, lb1, rows_per_tile=32)

    inner0 = _lateral(x0, iw0, ib0, rows_per_tile=32, td=inner1)
    r0 = _smooth(inner0, lw0, lb0, rows_per_tile=32)

    pool = r2[:, :, ::2, ::2]
    return (r0, r1, r2, pool)


# speedup vs baseline: 1.4453x; 1.1195x over previous
import jax
import jax.numpy as jnp
from jax import lax
from jax.experimental import pallas as pl
from jax.experimental.pallas import tpu as pltpu

BF16 = jnp.bfloat16
F32 = jnp.float32

_CPARAMS = dict(
    compiler_params=pltpu.CompilerParams(
        dimension_semantics=("parallel", "parallel"),
        vmem_limit_bytes=100 * 1024 * 1024,
    ),
)


def _lat_dot(x, w, b):
    acc = lax.dot_general(x.astype(BF16), w, (((0,), (0,)), ((), ())),
                          preferred_element_type=F32)
    return acc + b


def _up2x_rows(td, rows, w2, c):
    t = td.reshape(rows, w2, c)
    t = jnp.broadcast_to(t[:, :, None, :], (rows, w2, 2, c)).reshape(rows, 2 * w2, c)
    t = jnp.broadcast_to(t[:, None, :, :], (rows, 2, 2 * w2, c)).reshape(2 * rows, 2 * w2, c)
    return t.reshape(4 * rows * w2, c)


def _up2x_cols(row, w2, c):
    return jnp.broadcast_to(row[:, None, :], (w2, 2, c)).reshape(2 * w2, c)


def _conv3x3_acc(xpad_ref, w3_ref, acc_ref, th, w, c, cout):
    acc_ref[...] = jnp.zeros((th * w, cout), F32)
    for dy in range(3):
        for dx in range(3):
            patch = xpad_ref[dy:dy + th, dx:dx + w, :]
            acc_ref[...] += jnp.dot(
                patch.reshape(th * w, c), w3_ref[dy * 3 + dx],
                preferred_element_type=F32)


def _whole_body(H, W, C, x_ref, w1_ref, b1_ref, td_ref, w3_ref, b3_ref,
                r_ref, inner_ref, pool_ref, xpad_ref, acc_ref):
    lat = _lat_dot(x_ref[0], w1_ref[...], b1_ref[...])
    if td_ref is not None:
        lat = lat + _up2x_rows(td_ref[0].astype(F32), H // 2, W // 2, C)
    inner = lat.astype(BF16)
    inner_ref[0] = inner

    xpad_ref[:, 0, :] = jnp.zeros((H + 2, C), BF16)
    xpad_ref[:, W + 1, :] = jnp.zeros((H + 2, C), BF16)
    xpad_ref[0, 1:W + 1, :] = jnp.zeros((W, C), BF16)
    xpad_ref[H + 1, 1:W + 1, :] = jnp.zeros((W, C), BF16)
    xpad_ref[1:H + 1, 1:W + 1, :] = inner.reshape(H, W, C)

    _conv3x3_acc(xpad_ref, w3_ref, acc_ref, H, W, C, C)
    r_ref[0] = jnp.transpose(acc_ref[...]) + b3_ref[...]
    if pool_ref is not None:
        p = acc_ref[...].reshape(H // 2, 2, W // 2, 2, C)[:, 0, :, 0, :]
        pool_ref[0] = jnp.transpose(p.reshape((H // 2) * (W // 2), C)) \
            + b3_ref[...]


def _level_whole(x_nchw, iw, ib, lw, lb, td=None, pool=False):
    N, Cin, H, W = x_nchw.shape
    C = iw.shape[0]
    xm = x_nchw.reshape(N, Cin, H * W)
    w1 = jnp.transpose(iw.reshape(C, Cin)).astype(BF16)
    b1 = ib.reshape(1, C)
    w3 = jnp.transpose(lw, (2, 3, 1, 0)).reshape(9, C, C).astype(BF16)
    b3 = lb.reshape(C, 1)

    in_specs = [
        pl.BlockSpec((1, Cin, H * W), lambda n: (n, 0, 0)),
        pl.BlockSpec((Cin, C), lambda n: (0, 0)),
        pl.BlockSpec((1, C), lambda n: (0, 0)),
    ]
    args = [xm, w1, b1]
    if td is not None:
        in_specs.append(pl.BlockSpec((1, (H // 2) * (W // 2), C), lambda n: (n, 0, 0)))
        args.append(td)
    in_specs += [
        pl.BlockSpec((9, C, C), lambda n: (0, 0, 0)),
        pl.BlockSpec((C, 1), lambda n: (0, 0)),
    ]
    args += [w3, b3]

    out_shape = [
        jax.ShapeDtypeStruct((N, C, H * W), F32),
        jax.ShapeDtypeStruct((N, H * W, C), BF16),
    ]
    out_specs = [
        pl.BlockSpec((1, C, H * W), lambda n: (n, 0, 0)),
        pl.BlockSpec((1, H * W, C), lambda n: (n, 0, 0)),
    ]
    if pool:
        out_shape.append(jax.ShapeDtypeStruct((N, C, (H // 2) * (W // 2)), F32))
        out_specs.append(pl.BlockSpec((1, C, (H // 2) * (W // 2)), lambda n: (n, 0, 0)))

    def kfn(*refs):
        it = iter(refs)
        x_ref = next(it); w1_ref = next(it); b1_ref = next(it)
        td_ref = next(it) if td is not None else None
        w3_ref = next(it); b3_ref = next(it)
        r_ref = next(it); inner_ref = next(it)
        pool_ref = next(it) if pool else None
        xpad_ref = next(it); acc_ref = next(it)
        _whole_body(H, W, C, x_ref, w1_ref, b1_ref, td_ref, w3_ref, b3_ref,
                    r_ref, inner_ref, pool_ref, xpad_ref, acc_ref)

    outs = pl.pallas_call(
        kfn,
        out_shape=out_shape,
        grid=(N,),
        in_specs=in_specs,
        out_specs=out_specs,
        scratch_shapes=[
            pltpu.VMEM((H + 2, W + 2, C), BF16),
            pltpu.VMEM((H * W, C), F32),
        ],
        compiler_params=pltpu.CompilerParams(
            dimension_semantics=("parallel",),
            vmem_limit_bytes=100 * 1024 * 1024,
        ),
    )(*args)
    return outs


def _make_l0_kernel(TH, W, C, NT):
    def kfn(xm_ref, xt_ref, xb_ref, w1_ref, b1_ref,
            tdm_ref, tdt_ref, tdb_ref, w3_ref, b3_ref,
            r_ref, xpad_ref, acc_ref):
        t = pl.program_id(1)
        w2 = W // 2

        lat = _lat_dot(xm_ref[0], w1_ref[...], b1_ref[...])
        lat = lat + _up2x_rows(tdm_ref[0].astype(F32), TH // 2, w2, C)
        main = lat.astype(BF16).reshape(TH, W, C)

        top = _lat_dot(xt_ref[0], w1_ref[...], b1_ref[...])
        top = top + _up2x_cols(tdt_ref[0].astype(F32), w2, C)
        top = jnp.where(t > 0, top, 0.0).astype(BF16)

        bot = _lat_dot(xb_ref[0], w1_ref[...], b1_ref[...])
        bot = bot + _up2x_cols(tdb_ref[0].astype(F32), w2, C)
        bot = jnp.where(t < NT - 1, bot, 0.0).astype(BF16)

        xpad_ref[:, 0, :] = jnp.zeros((TH + 2, C), BF16)
        xpad_ref[:, W + 1, :] = jnp.zeros((TH + 2, C), BF16)
        xpad_ref[0, 1:W + 1, :] = top
        xpad_ref[TH + 1, 1:W + 1, :] = bot
        xpad_ref[1:TH + 1, 1:W + 1, :] = main

        _conv3x3_acc(xpad_ref, w3_ref, acc_ref, TH, W, C, C)
        r_ref[0] = jnp.transpose(acc_ref[...]) + b3_ref[...]
    return kfn


def _level0(x_nchw, iw, ib, lw, lb, td, TH=32):
    N, Cin, H, W = x_nchw.shape
    C = iw.shape[0]
    h2, w2 = H // 2, W // 2
    NT = H // TH
    xm = x_nchw.reshape(N, Cin, H * W)
    w1 = jnp.transpose(iw.reshape(C, Cin)).astype(BF16)
    b1 = ib.reshape(1, C)
    w3 = jnp.transpose(lw, (2, 3, 1, 0)).reshape(9, C, C).astype(BF16)
    b3 = lb.reshape(C, 1)

    TH2 = TH // 2
    in_specs = [
        pl.BlockSpec((1, Cin, TH * W), lambda n, t: (n, 0, t)),
        pl.BlockSpec((1, Cin, W), lambda n, t: (n, 0, jnp.maximum(t * TH - 1, 0))),
        pl.BlockSpec((1, Cin, W), lambda n, t: (n, 0, jnp.minimum(t * TH + TH, H - 1))),
        pl.BlockSpec((Cin, C), lambda n, t: (0, 0)),
        pl.BlockSpec((1, C), lambda n, t: (0, 0)),
        pl.BlockSpec((1, TH2 * w2, C), lambda n, t: (n, t, 0)),
        pl.BlockSpec((1, w2, C), lambda n, t: (n, jnp.maximum(t * TH2 - 1, 0), 0)),
        pl.BlockSpec((1, w2, C), lambda n, t: (n, jnp.minimum(t * TH2 + TH2, h2 - 1), 0)),
        pl.BlockSpec((9, C, C), lambda n, t: (0, 0, 0)),
        pl.BlockSpec((C, 1), lambda n, t: (0, 0)),
    ]
    out = pl.pallas_call(
        _make_l0_kernel(TH, W, C, NT),
        out_shape=jax.ShapeDtypeStruct((N, C, H * W), F32),
        grid=(N, NT),
        in_specs=in_specs,
        out_specs=pl.BlockSpec((1, C, TH * W), lambda n, t: (n, 0, t)),
        scratch_shapes=[
            pltpu.VMEM((TH + 2, W + 2, C), BF16),
            pltpu.VMEM((TH * W, C), F32),
        ],
        **_CPARAMS,
    )(xm, xm, xm, w1, b1, td, td, td, w3, b3)
    return out


def kernel(x0, x1, x2, iw0, ib0, lw0, lb0, iw1, ib1, lw1, lb1, iw2, ib2, lw2, lb2):
    N = x0.shape[0]
    C = iw0.shape[0]

    r2f, inner2, poolf = _level_whole(x2, iw2, ib2, lw2, lb2, pool=True)

    r1f, inner1 = _level_whole(x1, iw1, ib1, lw1, lb1, td=inner2)

    r0f = _level0(x0, iw0, ib0, lw0, lb0, td=inner1)

    H0 = x0.shape[2]
    H1 = x1.shape[2]
    H2 = x2.shape[2]
    r0 = r0f.reshape(N, C, H0, H0)
    r1 = r1f.reshape(N, C, H1, H1)
    r2 = r2f.reshape(N, C, H2, H2)
    pool = poolf.reshape(N, C, H2 // 2, H2 // 2)
    return (r0, r1, r2, pool)


# speedup vs baseline: 1.5020x; 1.0393x over previous
import jax
import jax.numpy as jnp
from jax import lax
from jax.experimental import pallas as pl
from jax.experimental.pallas import tpu as pltpu

BF16 = jnp.bfloat16
F32 = jnp.float32

_CPARAMS = dict(
    compiler_params=pltpu.CompilerParams(
        dimension_semantics=("parallel", "arbitrary"),
        vmem_limit_bytes=100 * 1024 * 1024,
    ),
)


def _lat_dot(x, w, b):
    acc = lax.dot_general(x.astype(BF16), w, (((0,), (0,)), ((), ())),
                          preferred_element_type=F32)
    return acc + b


def _up2x_rows(td, rows, w2, c):
    t = td.reshape(rows, w2, c)
    t = jnp.broadcast_to(t[:, :, None, :], (rows, w2, 2, c)).reshape(rows, 2 * w2, c)
    t = jnp.broadcast_to(t[:, None, :, :], (rows, 2, 2 * w2, c)).reshape(2 * rows, 2 * w2, c)
    return t.reshape(4 * rows * w2, c)


def _up2x_cols(row, w2, c):
    return jnp.broadcast_to(row[:, None, :], (w2, 2, c)).reshape(2 * w2, c)


def _conv3x3_acc(val, w3_ref, bc_ref, bl_ref, br_ref, acc_ref, th, w, c):
    m2 = (th + 2) * w
    bc_ref[...] = val
    xix = lax.broadcasted_iota(jnp.int32, (m2, 1), 0) % w
    zrow = jnp.zeros((1, c), BF16)
    vl = jnp.concatenate([val[1:], zrow], axis=0)
    bl_ref[...] = jnp.where(xix == w - 1, zrow, vl)
    vr = jnp.concatenate([zrow, val[:-1]], axis=0)
    br_ref[...] = jnp.where(xix == 0, zrow, vr)
    acc_ref[...] = jnp.zeros((th * w, c), F32)
    for dy in range(3):
        s = pl.ds(dy * w, th * w)
        lhs = jnp.concatenate([br_ref[s], bc_ref[s], bl_ref[s]], axis=1)
        acc_ref[...] += jnp.dot(lhs, w3_ref[dy], preferred_element_type=F32)


def _whole_body(H, W, C, x_ref, w1_ref, b1_ref, td_ref, w3_ref, b3_ref,
                r_ref, inner_ref, pool_ref, bc_ref, bl_ref, br_ref, acc_ref):
    lat = _lat_dot(x_ref[0], w1_ref[...], b1_ref[...])
    if td_ref is not None:
        lat = lat + _up2x_rows(td_ref[0].astype(F32), H // 2, W // 2, C)
    inner = lat.astype(BF16)
    inner_ref[0] = inner

    zr = jnp.zeros((W, C), BF16)
    val = jnp.concatenate([zr, inner, zr], axis=0)

    _conv3x3_acc(val, w3_ref, bc_ref, bl_ref, br_ref, acc_ref, H, W, C)
    r_ref[0] = jnp.transpose(acc_ref[...]) + b3_ref[...]
    if pool_ref is not None:
        p = acc_ref[...].reshape(H // 2, 2, W // 2, 2, C)[:, 0, :, 0, :]
        pool_ref[0] = jnp.transpose(p.reshape((H // 2) * (W // 2), C)) \
            + b3_ref[...]


def _level_whole(x_nchw, iw, ib, lw, lb, td=None, pool=False):
    N, Cin, H, W = x_nchw.shape
    C = iw.shape[0]
    xm = x_nchw.reshape(N, Cin, H * W)
    w1 = jnp.transpose(iw.reshape(C, Cin)).astype(BF16)
    b1 = ib.reshape(1, C)
    w3 = jnp.transpose(lw, (2, 3, 1, 0)).reshape(3, 3 * C, C).astype(BF16)
    b3 = lb.reshape(C, 1)

    in_specs = [
        pl.BlockSpec((1, Cin, H * W), lambda n: (n, 0, 0)),
        pl.BlockSpec((Cin, C), lambda n: (0, 0)),
        pl.BlockSpec((1, C), lambda n: (0, 0)),
    ]
    args = [xm, w1, b1]
    if td is not None:
        in_specs.append(pl.BlockSpec((1, (H // 2) * (W // 2), C), lambda n: (n, 0, 0)))
        args.append(td)
    in_specs += [
        pl.BlockSpec((3, 3 * C, C), lambda n: (0, 0, 0)),
        pl.BlockSpec((C, 1), lambda n: (0, 0)),
    ]
    args += [w3, b3]

    out_shape = [
        jax.ShapeDtypeStruct((N, C, H * W), F32),
        jax.ShapeDtypeStruct((N, H * W, C), BF16),
    ]
    out_specs = [
        pl.BlockSpec((1, C, H * W), lambda n: (n, 0, 0)),
        pl.BlockSpec((1, H * W, C), lambda n: (n, 0, 0)),
    ]
    if pool:
        out_shape.append(jax.ShapeDtypeStruct((N, C, (H // 2) * (W // 2)), F32))
        out_specs.append(pl.BlockSpec((1, C, (H // 2) * (W // 2)), lambda n: (n, 0, 0)))

    def kfn(*refs):
        it = iter(refs)
        x_ref = next(it); w1_ref = next(it); b1_ref = next(it)
        td_ref = next(it) if td is not None else None
        w3_ref = next(it); b3_ref = next(it)
        r_ref = next(it); inner_ref = next(it)
        pool_ref = next(it) if pool else None
        bc_ref = next(it); bl_ref = next(it); br_ref = next(it)
        acc_ref = next(it)
        _whole_body(H, W, C, x_ref, w1_ref, b1_ref, td_ref, w3_ref, b3_ref,
                    r_ref, inner_ref, pool_ref, bc_ref, bl_ref, br_ref, acc_ref)

    outs = pl.pallas_call(
        kfn,
        out_shape=out_shape,
        grid=(N,),
        in_specs=in_specs,
        out_specs=out_specs,
        scratch_shapes=[
            pltpu.VMEM(((H + 2) * W, C), BF16),
            pltpu.VMEM(((H + 2) * W, C), BF16),
            pltpu.VMEM(((H + 2) * W, C), BF16),
            pltpu.VMEM((H * W, C), F32),
        ],
        compiler_params=pltpu.CompilerParams(
            dimension_semantics=("parallel",),
            vmem_limit_bytes=100 * 1024 * 1024,
        ),
    )(*args)
    return outs


def _make_l0_kernel(TH, W, C, NT):
    def kfn(xm_ref, xt_ref, xb_ref, w1_ref, b1_ref,
            tdm_ref, tdt_ref, tdb_ref, w3_ref, b3_ref,
            r_ref, bc_ref, bl_ref, br_ref, acc_ref):
        t = pl.program_id(1)
        w2 = W // 2

        lat = _lat_dot(xm_ref[0], w1_ref[...], b1_ref[...])
        lat = lat + _up2x_rows(tdm_ref[0].astype(F32), TH // 2, w2, C)
        main = lat.astype(BF16)

        top = _lat_dot(xt_ref[0], w1_ref[...], b1_ref[...])
        top = top + _up2x_cols(tdt_ref[0].astype(F32), w2, C)
        top = jnp.where(t > 0, top, 0.0).astype(BF16)

        bot = _lat_dot(xb_ref[0], w1_ref[...], b1_ref[...])
        bot = bot + _up2x_cols(tdb_ref[0].astype(F32), w2, C)
        bot = jnp.where(t < NT - 1, bot, 0.0).astype(BF16)

        val = jnp.concatenate([top, main, bot], axis=0)
        _conv3x3_acc(val, w3_ref, bc_ref, bl_ref, br_ref, acc_ref, TH, W, C)
        r_ref[0] = jnp.transpose(acc_ref[...]) + b3_ref[...]
    return kfn


def _level0(x_nchw, iw, ib, lw, lb, td, TH=32):
    N, Cin, H, W = x_nchw.shape
    C = iw.shape[0]
    h2, w2 = H // 2, W // 2
    NT = H // TH
    xm = x_nchw.reshape(N, Cin, H * W)
    w1 = jnp.transpose(iw.reshape(C, Cin)).astype(BF16)
    b1 = ib.reshape(1, C)
    w3 = jnp.transpose(lw, (2, 3, 1, 0)).reshape(3, 3 * C, C).astype(BF16)
    b3 = lb.reshape(C, 1)

    TH2 = TH // 2
    in_specs = [
        pl.BlockSpec((1, Cin, TH * W), lambda n, t: (n, 0, t)),
        pl.BlockSpec((1, Cin, W), lambda n, t: (n, 0, jnp.maximum(t * TH - 1, 0))),
        pl.BlockSpec((1, Cin, W), lambda n, t: (n, 0, jnp.minimum(t * TH + TH, H - 1))),
        pl.BlockSpec((Cin, C), lambda n, t: (0, 0)),
        pl.BlockSpec((1, C), lambda n, t: (0, 0)),
        pl.BlockSpec((1, TH2 * w2, C), lambda n, t: (n, t, 0)),
        pl.BlockSpec((1, w2, C), lambda n, t: (n, jnp.maximum(t * TH2 - 1, 0), 0)),
        pl.BlockSpec((1, w2, C), lambda n, t: (n, jnp.minimum(t * TH2 + TH2, h2 - 1), 0)),
        pl.BlockSpec((3, 3 * C, C), lambda n, t: (0, 0, 0)),
        pl.BlockSpec((C, 1), lambda n, t: (0, 0)),
    ]
    out = pl.pallas_call(
        _make_l0_kernel(TH, W, C, NT),
        out_shape=jax.ShapeDtypeStruct((N, C, H * W), F32),
        grid=(N, NT),
        in_specs=in_specs,
        out_specs=pl.BlockSpec((1, C, TH * W), lambda n, t: (n, 0, t)),
        scratch_shapes=[
            pltpu.VMEM(((TH + 2) * W, C), BF16),
            pltpu.VMEM(((TH + 2) * W, C), BF16),
            pltpu.VMEM(((TH + 2) * W, C), BF16),
            pltpu.VMEM((TH * W, C), F32),
        ],
        **_CPARAMS,
    )(xm, xm, xm, w1, b1, td, td, td, w3, b3)
    return out


def kernel(x0, x1, x2, iw0, ib0, lw0, lb0, iw1, ib1, lw1, lb1, iw2, ib2, lw2, lb2):
    N = x0.shape[0]
    C = iw0.shape[0]

    r2f, inner2, poolf = _level_whole(x2, iw2, ib2, lw2, lb2, pool=True)

    r1f, inner1 = _level_whole(x1, iw1, ib1, lw1, lb1, td=inner2)

    r0f = _level0(x0, iw0, ib0, lw0, lb0, td=inner1)

    H0 = x0.shape[2]
    H1 = x1.shape[2]
    H2 = x2.shape[2]
    r0 = r0f.reshape(N, C, H0, H0)
    r1 = r1f.reshape(N, C, H1, H1)
    r2 = r2f.reshape(N, C, H2, H2)
    pool = poolf.reshape(N, C, H2 // 2, H2 // 2)
    return (r0, r1, r2, pool)


# speedup vs baseline: 2.0601x; 1.3715x over previous
import jax
import jax.numpy as jnp
from jax import lax
from jax.experimental import pallas as pl
from jax.experimental.pallas import tpu as pltpu

BF16 = jnp.bfloat16
F32 = jnp.float32


def _up2x_rows(td, rows, w2, c):
    t = td.reshape(rows, w2, c)
    t = jnp.broadcast_to(t[:, :, None, :], (rows, w2, 2, c)).reshape(rows, 2 * w2, c)
    t = jnp.broadcast_to(t[:, None, :, :], (rows, 2, 2 * w2, c)).reshape(2 * rows, 2 * w2, c)
    return t.reshape(4 * rows * w2, c)


def _up2x_cols(row, w2, c):
    return jnp.broadcast_to(row[:, None, :], (w2, 2, c)).reshape(2 * w2, c)


def _conv3x3_acc(val, w3_ref, bc_ref, bl_ref, br_ref, acc_ref, th, w, c):
    m2 = (th + 2) * w
    bc_ref[...] = val
    xix = lax.broadcasted_iota(jnp.int32, (m2, 1), 0) % w
    zrow = jnp.zeros((1, c), BF16)
    vl = jnp.concatenate([val[1:], zrow], axis=0)
    bl_ref[...] = jnp.where(xix == w - 1, zrow, vl)
    vr = jnp.concatenate([zrow, val[:-1]], axis=0)
    br_ref[...] = jnp.where(xix == 0, zrow, vr)
    acc_ref[...] = jnp.zeros((th * w, c), F32)
    for dy in range(3):
        s = pl.ds(dy * w, th * w)
        lhs = jnp.concatenate([br_ref[s], bc_ref[s], bl_ref[s]], axis=1)
        acc_ref[...] += jnp.dot(lhs, w3_ref[dy], preferred_element_type=F32)


def _whole_body(H, W, C, x_ref, w1_ref, b1_ref, td_ref, w3_ref, b3_ref,
                r_ref, inner_ref, pool_ref, bc_ref, bl_ref, br_ref, acc_ref):
    lat = jnp.dot(x_ref[0].astype(BF16), w1_ref[...],
                  preferred_element_type=F32) + b1_ref[...]
    if td_ref is not None:
        lat = lat + _up2x_rows(td_ref[0].astype(F32), H // 2, W // 2, C)
    inner = lat.astype(BF16)
    inner_ref[0] = inner

    zr = jnp.zeros((W, C), BF16)
    val = jnp.concatenate([zr, inner, zr], axis=0)

    _conv3x3_acc(val, w3_ref, bc_ref, bl_ref, br_ref, acc_ref, H, W, C)
    r_ref[0] = acc_ref[...] + b3_ref[...]
    if pool_ref is not None:
        p = acc_ref[...].reshape(H // 2, 2, W // 2, 2, C)[:, 0, :, 0, :]
        pool_ref[0] = p.reshape((H // 2) * (W // 2), C) + b3_ref[...]


def _level_whole(x_hwc, iw, ib, lw, lb, td=None, pool=False):
    N, HW, Cin = x_hwc.shape
    C = iw.shape[0]
    H = W = int(HW ** 0.5)
    assert H * W == HW
    w1 = jnp.transpose(iw.reshape(C, Cin)).astype(BF16)
    b1 = ib.reshape(1, C)
    w3 = jnp.transpose(lw, (2, 3, 1, 0)).reshape(3, 3 * C, C).astype(BF16)
    b3 = lb.reshape(1, C)

    in_specs = [
        pl.BlockSpec((1, HW, Cin), lambda n: (n, 0, 0)),
        pl.BlockSpec((Cin, C), lambda n: (0, 0)),
        pl.BlockSpec((1, C), lambda n: (0, 0)),
    ]
    args = [x_hwc, w1, b1]
    if td is not None:
        in_specs.append(pl.BlockSpec((1, HW // 4, C), lambda n: (n, 0, 0)))
        args.append(td)
    in_specs += [
        pl.BlockSpec((3, 3 * C, C), lambda n: (0, 0, 0)),
        pl.BlockSpec((1, C), lambda n: (0, 0)),
    ]
    args += [w3, b3]

    out_shape = [
        jax.ShapeDtypeStruct((N, HW, C), F32),
        jax.ShapeDtypeStruct((N, HW, C), BF16),
    ]
    out_specs = [
        pl.BlockSpec((1, HW, C), lambda n: (n, 0, 0)),
        pl.BlockSpec((1, HW, C), lambda n: (n, 0, 0)),
    ]
    if pool:
        out_shape.append(jax.ShapeDtypeStruct((N, HW // 4, C), F32))
        out_specs.append(pl.BlockSpec((1, HW // 4, C), lambda n: (n, 0, 0)))

    def kfn(*refs):
        it = iter(refs)
        x_ref = next(it); w1_ref = next(it); b1_ref = next(it)
        td_ref = next(it) if td is not None else None
        w3_ref = next(it); b3_ref = next(it)
        r_ref = next(it); inner_ref = next(it)
        pool_ref = next(it) if pool else None
        bc_ref = next(it); bl_ref = next(it); br_ref = next(it)
        acc_ref = next(it)
        _whole_body(H, W, C, x_ref, w1_ref, b1_ref, td_ref, w3_ref, b3_ref,
                    r_ref, inner_ref, pool_ref, bc_ref, bl_ref, br_ref, acc_ref)

    outs = pl.pallas_call(
        kfn,
        out_shape=out_shape,
        grid=(N,),
        in_specs=in_specs,
        out_specs=out_specs,
        scratch_shapes=[
            pltpu.VMEM(((H + 2) * W, C), BF16),
            pltpu.VMEM(((H + 2) * W, C), BF16),
            pltpu.VMEM(((H + 2) * W, C), BF16),
            pltpu.VMEM((HW, C), F32),
        ],
        compiler_params=pltpu.CompilerParams(
            dimension_semantics=("parallel",),
            vmem_limit_bytes=100 * 1024 * 1024,
        ),
    )(*args)
    return outs


def _make_l0_kernel(TH, W, C, NT):
    def kfn(xm_ref, xt_ref, xb_ref, w1_ref, b1_ref,
            tdm_ref, tdt_ref, tdb_ref, w3_ref, b3_ref,
            r_ref, bc_ref, bl_ref, br_ref, acc_ref):
        t = pl.program_id(1)
        w2 = W // 2
        cin = xm_ref.shape[1]

        def lat_dot(x2d):
            return lax.dot_general(x2d.astype(BF16), w1_ref[...],
                                   (((0,), (0,)), ((), ())),
                                   preferred_element_type=F32) + b1_ref[...]

        lat = lat_dot(xm_ref[0].reshape(cin, TH * W))
        lat = lat + _up2x_rows(tdm_ref[0].astype(F32), TH // 2, w2, C)
        main = lat.astype(BF16)

        top = lat_dot(xt_ref[0, :, 7, :])
        top = top + _up2x_cols(tdt_ref[0].astype(F32), w2, C)
        top = jnp.where(t > 0, top, 0.0).astype(BF16)

        bot = lat_dot(xb_ref[0, :, 0, :])
        bot = bot + _up2x_cols(tdb_ref[0].astype(F32), w2, C)
        bot = jnp.where(t < NT - 1, bot, 0.0).astype(BF16)

        val = jnp.concatenate([top, main, bot], axis=0)
        _conv3x3_acc(val, w3_ref, bc_ref, bl_ref, br_ref, acc_ref, TH, W, C)
        r_ref[0] = acc_ref[...] + b3_ref[...]
    return kfn


def _level0(x_nchw, iw, ib, lw, lb, td, TH=32):
    N, Cin, H, W = x_nchw.shape
    C = iw.shape[0]
    h2, w2 = H // 2, W // 2
    NT = H // TH
    w1 = jnp.transpose(iw.reshape(C, Cin)).astype(BF16)
    b1 = ib.reshape(1, C)
    w3 = jnp.transpose(lw, (2, 3, 1, 0)).reshape(3, 3 * C, C).astype(BF16)
    b3 = lb.reshape(1, C)

    TH2 = TH // 2
    in_specs = [
        pl.BlockSpec((1, Cin, TH, W), lambda n, t: (n, 0, t, 0)),
        pl.BlockSpec((1, Cin, 8, W),
                     lambda n, t: (n, 0, jnp.maximum(t * (TH // 8) - 1, 0), 0)),
        pl.BlockSpec((1, Cin, 8, W),
                     lambda n, t: (n, 0, jnp.minimum((t + 1) * (TH // 8), H // 8 - 1), 0)),
        pl.BlockSpec((Cin, C), lambda n, t: (0, 0)),
        pl.BlockSpec((1, C), lambda n, t: (0, 0)),
        pl.BlockSpec((1, TH2 * w2, C), lambda n, t: (n, t, 0)),
        pl.BlockSpec((1, w2, C), lambda n, t: (n, jnp.maximum(t * TH2 - 1, 0), 0)),
        pl.BlockSpec((1, w2, C), lambda n, t: (n, jnp.minimum(t * TH2 + TH2, h2 - 1), 0)),
        pl.BlockSpec((3, 3 * C, C), lambda n, t: (0, 0, 0)),
        pl.BlockSpec((1, C), lambda n, t: (0, 0)),
    ]
    out = pl.pallas_call(
        _make_l0_kernel(TH, W, C, NT),
        out_shape=jax.ShapeDtypeStruct((N, H * W, C), F32),
        grid=(N, NT),
        in_specs=in_specs,
        out_specs=pl.BlockSpec((1, TH * W, C), lambda n, t: (n, t, 0)),
        scratch_shapes=[
            pltpu.VMEM(((TH + 2) * W, C), BF16),
            pltpu.VMEM(((TH + 2) * W, C), BF16),
            pltpu.VMEM(((TH + 2) * W, C), BF16),
            pltpu.VMEM((TH * W, C), F32),
        ],
        compiler_params=pltpu.CompilerParams(
            dimension_semantics=("parallel", "arbitrary"),
            vmem_limit_bytes=100 * 1024 * 1024,
        ),
    )(x_nchw, x_nchw, x_nchw, w1, b1, td, td, td, w3, b3)
    return out


def _to_nchw(r_hwc, N, C, H, W):
    return jnp.transpose(r_hwc, (0, 2, 1)).reshape(N, C, H, W)


def kernel(x0, x1, x2, iw0, ib0, lw0, lb0, iw1, ib1, lw1, lb1, iw2, ib2, lw2, lb2):
    N = x0.shape[0]
    C = iw0.shape[0]
    H0, H1, H2 = x0.shape[2], x1.shape[2], x2.shape[2]

    xh1 = jnp.transpose(x1, (0, 2, 3, 1)).reshape(N, H1 * H1, x1.shape[1])
    xh2 = jnp.transpose(x2, (0, 2, 3, 1)).reshape(N, H2 * H2, x2.shape[1])

    r2f, inner2, poolf = _level_whole(xh2, iw2, ib2, lw2, lb2, pool=True)

    r1f, inner1 = _level_whole(xh1, iw1, ib1, lw1, lb1, td=inner2)

    r0f = _level0(x0, iw0, ib0, lw0, lb0, td=inner1)

    r0 = _to_nchw(r0f, N, C, H0, H0)
    r1 = _to_nchw(r1f, N, C, H1, H1)
    r2 = _to_nchw(r2f, N, C, H2, H2)
    pool = _to_nchw(poolf, N, C, H2 // 2, H2 // 2)
    return (r0, r1, r2, pool)


# speedup vs baseline: 2.5145x; 1.2206x over previous
import jax
import jax.numpy as jnp
from jax import lax
from jax.experimental import pallas as pl
from jax.experimental.pallas import tpu as pltpu

BF16 = jnp.bfloat16
F32 = jnp.float32


def _up2x_rows(td, rows, w2, c):
    t = td.reshape(rows, w2, c)
    t = jnp.broadcast_to(t[:, :, None, :], (rows, w2, 2, c)).reshape(rows, 2 * w2, c)
    t = jnp.broadcast_to(t[:, None, :, :], (rows, 2, 2 * w2, c)).reshape(2 * rows, 2 * w2, c)
    return t.reshape(4 * rows * w2, c)


def _up2x_cols(row, w2, c):
    return jnp.broadcast_to(row[:, None, :], (w2, 2, c)).reshape(2 * w2, c)


def _conv3x3_acc(val, w3_ref, bc_ref, bl_ref, br_ref, acc_ref, th, w, c):
    m2 = (th + 2) * w
    bc_ref[...] = val
    xix = lax.broadcasted_iota(jnp.int32, (m2, 1), 0) % w
    zrow = jnp.zeros((1, c), BF16)
    vl = jnp.concatenate([val[1:], zrow], axis=0)
    bl_ref[...] = jnp.where(xix == w - 1, zrow, vl)
    vr = jnp.concatenate([zrow, val[:-1]], axis=0)
    br_ref[...] = jnp.where(xix == 0, zrow, vr)
    acc_ref[...] = jnp.zeros((th * w, c), F32)
    for dy in range(3):
        s = pl.ds(dy * w, th * w)
        lhs = jnp.concatenate([br_ref[s], bc_ref[s], bl_ref[s]], axis=1)
        acc_ref[...] += jnp.dot(lhs, w3_ref[dy], preferred_element_type=F32)


def _whole_body(H, W, C, x_ref, w1_ref, b1_ref, td_ref, w3_ref, b3_ref,
                r_ref, inner_ref, pool_ref, bc_ref, bl_ref, br_ref, acc_ref):
    lat = jnp.dot(x_ref[0].astype(BF16), w1_ref[...],
                  preferred_element_type=F32) + b1_ref[...]
    if td_ref is not None:
        lat = lat + _up2x_rows(td_ref[0].astype(F32), H // 2, W // 2, C)
    inner = lat.astype(BF16)
    inner_ref[0] = inner

    zr = jnp.zeros((W, C), BF16)
    val = jnp.concatenate([zr, inner, zr], axis=0)

    _conv3x3_acc(val, w3_ref, bc_ref, bl_ref, br_ref, acc_ref, H, W, C)
    r_ref[0] = acc_ref[...] + b3_ref[...]
    if pool_ref is not None:
        p = acc_ref[...].reshape(H // 2, 2, W // 2, 2, C)[:, 0, :, 0, :]
        pool_ref[0] = p.reshape((H // 2) * (W // 2), C) + b3_ref[...]


def _level_whole(x_hwc, iw, ib, lw, lb, td=None, pool=False):
    N, HW, Cin = x_hwc.shape
    C = iw.shape[0]
    H = W = int(HW ** 0.5)
    assert H * W == HW
    w1 = jnp.transpose(iw.reshape(C, Cin)).astype(BF16)
    b1 = ib.reshape(1, C)
    w3 = jnp.transpose(lw, (2, 3, 1, 0)).reshape(3, 3 * C, C).astype(BF16)
    b3 = lb.reshape(1, C)

    in_specs = [
        pl.BlockSpec((1, HW, Cin), lambda n: (n, 0, 0)),
        pl.BlockSpec((Cin, C), lambda n: (0, 0)),
        pl.BlockSpec((1, C), lambda n: (0, 0)),
    ]
    args = [x_hwc, w1, b1]
    if td is not None:
        in_specs.append(pl.BlockSpec((1, HW // 4, C), lambda n: (n, 0, 0)))
        args.append(td)
    in_specs += [
        pl.BlockSpec((3, 3 * C, C), lambda n: (0, 0, 0)),
        pl.BlockSpec((1, C), lambda n: (0, 0)),
    ]
    args += [w3, b3]

    out_shape = [
        jax.ShapeDtypeStruct((N, HW, C), F32),
        jax.ShapeDtypeStruct((N, HW, C), BF16),
    ]
    out_specs = [
        pl.BlockSpec((1, HW, C), lambda n: (n, 0, 0)),
        pl.BlockSpec((1, HW, C), lambda n: (n, 0, 0)),
    ]
    if pool:
        out_shape.append(jax.ShapeDtypeStruct((N, HW // 4, C), F32))
        out_specs.append(pl.BlockSpec((1, HW // 4, C), lambda n: (n, 0, 0)))

    def kfn(*refs):
        it = iter(refs)
        x_ref = next(it); w1_ref = next(it); b1_ref = next(it)
        td_ref = next(it) if td is not None else None
        w3_ref = next(it); b3_ref = next(it)
        r_ref = next(it); inner_ref = next(it)
        pool_ref = next(it) if pool else None
        bc_ref = next(it); bl_ref = next(it); br_ref = next(it)
        acc_ref = next(it)
        _whole_body(H, W, C, x_ref, w1_ref, b1_ref, td_ref, w3_ref, b3_ref,
                    r_ref, inner_ref, pool_ref, bc_ref, bl_ref, br_ref, acc_ref)

    outs = pl.pallas_call(
        kfn,
        out_shape=out_shape,
        grid=(N,),
        in_specs=in_specs,
        out_specs=out_specs,
        scratch_shapes=[
            pltpu.VMEM(((H + 2) * W, C), BF16),
            pltpu.VMEM(((H + 2) * W, C), BF16),
            pltpu.VMEM(((H + 2) * W, C), BF16),
            pltpu.VMEM((HW, C), F32),
        ],
        compiler_params=pltpu.CompilerParams(
            dimension_semantics=("parallel",),
            vmem_limit_bytes=100 * 1024 * 1024,
        ),
    )(*args)
    return outs


def _make_l0_kernel(TH, W, C, NT):
    def kfn(xm_ref, xt_ref, xb_ref, w1_ref, b1_ref,
            tdm_ref, tdt_ref, tdb_ref, w3_ref, b3_ref,
            r_ref, bc_ref, bl_ref, br_ref, acc_ref):
        t = pl.program_id(1)
        w2 = W // 2
        cin = xm_ref.shape[1]

        def lat_dot(x2d):
            return lax.dot_general(x2d.astype(BF16), w1_ref[...],
                                   (((0,), (0,)), ((), ())),
                                   preferred_element_type=F32) + b1_ref[...]

        lat = lat_dot(xm_ref[0].reshape(cin, TH * W))
        lat = lat + _up2x_rows(tdm_ref[0].astype(F32), TH // 2, w2, C)
        main = lat.astype(BF16)

        top = lat_dot(xt_ref[0, :, 7, :])
        top = top + _up2x_cols(tdt_ref[0].astype(F32), w2, C)
        top = jnp.where(t > 0, top, 0.0).astype(BF16)

        bot = lat_dot(xb_ref[0, :, 0, :])
        bot = bot + _up2x_cols(tdb_ref[0].astype(F32), w2, C)
        bot = jnp.where(t < NT - 1, bot, 0.0).astype(BF16)

        val = jnp.concatenate([top, main, bot], axis=0)
        _conv3x3_acc(val, w3_ref, bc_ref, bl_ref, br_ref, acc_ref, TH, W, C)
        r = jnp.transpose(acc_ref[...]) + b3_ref[...]
        r_ref[0] = r.reshape(C, TH, W)
    return kfn


def _level0(x_nchw, iw, ib, lw, lb, td, TH=32):
    N, Cin, H, W = x_nchw.shape
    C = iw.shape[0]
    h2, w2 = H // 2, W // 2
    NT = H // TH
    w1 = jnp.transpose(iw.reshape(C, Cin)).astype(BF16)
    b1 = ib.reshape(1, C)
    w3 = jnp.transpose(lw, (2, 3, 1, 0)).reshape(3, 3 * C, C).astype(BF16)
    b3 = lb.reshape(C, 1)

    TH2 = TH // 2
    in_specs = [
        pl.BlockSpec((1, Cin, TH, W), lambda n, t: (n, 0, t, 0)),
        pl.BlockSpec((1, Cin, 8, W),
                     lambda n, t: (n, 0, jnp.maximum(t * (TH // 8) - 1, 0), 0)),
        pl.BlockSpec((1, Cin, 8, W),
                     lambda n, t: (n, 0, jnp.minimum((t + 1) * (TH // 8), H // 8 - 1), 0)),
        pl.BlockSpec((Cin, C), lambda n, t: (0, 0)),
        pl.BlockSpec((1, C), lambda n, t: (0, 0)),
        pl.BlockSpec((1, TH2 * w2, C), lambda n, t: (n, t, 0)),
        pl.BlockSpec((1, w2, C), lambda n, t: (n, jnp.maximum(t * TH2 - 1, 0), 0)),
        pl.BlockSpec((1, w2, C), lambda n, t: (n, jnp.minimum(t * TH2 + TH2, h2 - 1), 0)),
        pl.BlockSpec((3, 3 * C, C), lambda n, t: (0, 0, 0)),
        pl.BlockSpec((C, 1), lambda n, t: (0, 0)),
    ]
    out = pl.pallas_call(
        _make_l0_kernel(TH, W, C, NT),
        out_shape=jax.ShapeDtypeStruct((N, C, H, W), F32),
        grid=(N, NT),
        in_specs=in_specs,
        out_specs=pl.BlockSpec((1, C, TH, W), lambda n, t: (n, 0, t, 0)),
        scratch_shapes=[
            pltpu.VMEM(((TH + 2) * W, C), BF16),
            pltpu.VMEM(((TH + 2) * W, C), BF16),
            pltpu.VMEM(((TH + 2) * W, C), BF16),
            pltpu.VMEM((TH * W, C), F32),
        ],
        compiler_params=pltpu.CompilerParams(
            dimension_semantics=("parallel", "arbitrary"),
            vmem_limit_bytes=100 * 1024 * 1024,
        ),
    )(x_nchw, x_nchw, x_nchw, w1, b1, td, td, td, w3, b3)
    return out


def _to_nchw(r_hwc, N, C, H, W):
    return jnp.transpose(r_hwc, (0, 2, 1)).reshape(N, C, H, W)


def kernel(x0, x1, x2, iw0, ib0, lw0, lb0, iw1, ib1, lw1, lb1, iw2, ib2, lw2, lb2):
    N = x0.shape[0]
    C = iw0.shape[0]
    H0, H1, H2 = x0.shape[2], x1.shape[2], x2.shape[2]

    xh1 = jnp.transpose(x1, (0, 2, 3, 1)).reshape(N, H1 * H1, x1.shape[1])
    xh2 = jnp.transpose(x2, (0, 2, 3, 1)).reshape(N, H2 * H2, x2.shape[1])

    r2f, inner2, poolf = _level_whole(xh2, iw2, ib2, lw2, lb2, pool=True)

    r1f, inner1 = _level_whole(xh1, iw1, ib1, lw1, lb1, td=inner2)

    r0f = _level0(x0, iw0, ib0, lw0, lb0, td=inner1)

    r0 = r0f
    r1 = _to_nchw(r1f, N, C, H1, H1)
    r2 = _to_nchw(r2f, N, C, H2, H2)
    pool = _to_nchw(poolf, N, C, H2 // 2, H2 // 2)
    return (r0, r1, r2, pool)


# speedup vs baseline: 2.5303x; 1.0063x over previous
import jax
import jax.numpy as jnp
from jax import lax
from jax.experimental import pallas as pl
from jax.experimental.pallas import tpu as pltpu

BF16 = jnp.bfloat16
F32 = jnp.float32


def _up2x_rows(td, rows, w2, c):
    t = td.reshape(rows, w2, c)
    t = jnp.broadcast_to(t[:, :, None, :], (rows, w2, 2, c)).reshape(rows, 2 * w2, c)
    t = jnp.broadcast_to(t[:, None, :, :], (rows, 2, 2 * w2, c)).reshape(2 * rows, 2 * w2, c)
    return t.reshape(4 * rows * w2, c)


def _up2x_cols(row, w2, c):
    return jnp.broadcast_to(row[:, None, :], (w2, 2, c)).reshape(2 * w2, c)


def _conv3x3_acc(val, w3_ref, bc_ref, bl_ref, br_ref, acc_ref, th, w, c):
    m2 = (th + 2) * w
    bc_ref[...] = val
    xix = lax.broadcasted_iota(jnp.int32, (m2, 1), 0) % w
    zrow = jnp.zeros((1, c), BF16)
    vl = jnp.concatenate([val[1:], zrow], axis=0)
    bl_ref[...] = jnp.where(xix == w - 1, zrow, vl)
    vr = jnp.concatenate([zrow, val[:-1]], axis=0)
    br_ref[...] = jnp.where(xix == 0, zrow, vr)
    acc_ref[...] = jnp.zeros((th * w, c), F32)
    for dy in range(3):
        s = pl.ds(dy * w, th * w)
        lhs = jnp.concatenate([br_ref[s], bc_ref[s], bl_ref[s]], axis=1)
        acc_ref[...] += jnp.dot(lhs, w3_ref[dy], preferred_element_type=F32)


def _whole_body(H, W, C, x_ref, w1_ref, b1_ref, td_ref, w3_ref, b3_ref,
                r_ref, inner_ref, pool_ref, bc_ref, bl_ref, br_ref, acc_ref):
    lat = jnp.dot(x_ref[0].astype(BF16), w1_ref[...],
                  preferred_element_type=F32) + b1_ref[...]
    if td_ref is not None:
        lat = lat + _up2x_rows(td_ref[0].astype(F32), H // 2, W // 2, C)
    inner = lat.astype(BF16)
    inner_ref[0] = inner

    zr = jnp.zeros((W, C), BF16)
    val = jnp.concatenate([zr, inner, zr], axis=0)

    _conv3x3_acc(val, w3_ref, bc_ref, bl_ref, br_ref, acc_ref, H, W, C)
    r_ref[0] = acc_ref[...] + b3_ref[...]
    if pool_ref is not None:
        p = acc_ref[...].reshape(H // 2, 2, W // 2, 2, C)[:, 0, :, 0, :]
        pool_ref[0] = p.reshape((H // 2) * (W // 2), C) + b3_ref[...]


def _level_whole(x_hwc, iw, ib, lw, lb, td=None, pool=False):
    N, HW, Cin = x_hwc.shape
    C = iw.shape[0]
    H = W = int(HW ** 0.5)
    assert H * W == HW
    w1 = jnp.transpose(iw.reshape(C, Cin)).astype(BF16)
    b1 = ib.reshape(1, C)
    w3 = jnp.transpose(lw, (2, 3, 1, 0)).reshape(3, 3 * C, C).astype(BF16)
    b3 = lb.reshape(1, C)

    in_specs = [
        pl.BlockSpec((1, HW, Cin), lambda n: (n, 0, 0)),
        pl.BlockSpec((Cin, C), lambda n: (0, 0)),
        pl.BlockSpec((1, C), lambda n: (0, 0)),
    ]
    args = [x_hwc, w1, b1]
    if td is not None:
        in_specs.append(pl.BlockSpec((1, HW // 4, C), lambda n: (n, 0, 0)))
        args.append(td)
    in_specs += [
        pl.BlockSpec((3, 3 * C, C), lambda n: (0, 0, 0)),
        pl.BlockSpec((1, C), lambda n: (0, 0)),
    ]
    args += [w3, b3]

    out_shape = [
        jax.ShapeDtypeStruct((N, HW, C), F32),
        jax.ShapeDtypeStruct((N, HW, C), BF16),
    ]
    out_specs = [
        pl.BlockSpec((1, HW, C), lambda n: (n, 0, 0)),
        pl.BlockSpec((1, HW, C), lambda n: (n, 0, 0)),
    ]
    if pool:
        out_shape.append(jax.ShapeDtypeStruct((N, HW // 4, C), F32))
        out_specs.append(pl.BlockSpec((1, HW // 4, C), lambda n: (n, 0, 0)))

    def kfn(*refs):
        it = iter(refs)
        x_ref = next(it); w1_ref = next(it); b1_ref = next(it)
        td_ref = next(it) if td is not None else None
        w3_ref = next(it); b3_ref = next(it)
        r_ref = next(it); inner_ref = next(it)
        pool_ref = next(it) if pool else None
        bc_ref = next(it); bl_ref = next(it); br_ref = next(it)
        acc_ref = next(it)
        _whole_body(H, W, C, x_ref, w1_ref, b1_ref, td_ref, w3_ref, b3_ref,
                    r_ref, inner_ref, pool_ref, bc_ref, bl_ref, br_ref, acc_ref)

    outs = pl.pallas_call(
        kfn,
        out_shape=out_shape,
        grid=(N,),
        in_specs=in_specs,
        out_specs=out_specs,
        scratch_shapes=[
            pltpu.VMEM(((H + 2) * W, C), BF16),
            pltpu.VMEM(((H + 2) * W, C), BF16),
            pltpu.VMEM(((H + 2) * W, C), BF16),
            pltpu.VMEM((HW, C), F32),
        ],
        compiler_params=pltpu.CompilerParams(
            dimension_semantics=("parallel",),
            vmem_limit_bytes=100 * 1024 * 1024,
        ),
    )(*args)
    return outs


def _make_l1_kernel(TH, W, C, NT):
    def kfn(xm_ref, xt_ref, xb_ref, w1_ref, b1_ref,
            tdm_ref, tdt_ref, tdb_ref, w3_ref, b3_ref,
            r_ref, inner_ref, bc_ref, bl_ref, br_ref, acc_ref):
        t = pl.program_id(1)
        w2 = W // 2

        def lat_dot(x2d):
            return jnp.dot(x2d.astype(BF16), w1_ref[...],
                           preferred_element_type=F32) + b1_ref[...]

        lat = lat_dot(xm_ref[0])
        lat = lat + _up2x_rows(tdm_ref[0].astype(F32), TH // 2, w2, C)
        main = lat.astype(BF16)
        inner_ref[0] = main

        top = lat_dot(xt_ref[0])
        top = top + _up2x_cols(tdt_ref[0].astype(F32), w2, C)
        top = jnp.where(t > 0, top, 0.0).astype(BF16)

        bot = lat_dot(xb_ref[0])
        bot = bot + _up2x_cols(tdb_ref[0].astype(F32), w2, C)
        bot = jnp.where(t < NT - 1, bot, 0.0).astype(BF16)

        val = jnp.concatenate([top, main, bot], axis=0)
        _conv3x3_acc(val, w3_ref, bc_ref, bl_ref, br_ref, acc_ref, TH, W, C)
        r_ref[0] = acc_ref[...] + b3_ref[...]
    return kfn


def _level1(x_hwc, iw, ib, lw, lb, td, TH=32):
    N, HW, Cin = x_hwc.shape
    C = iw.shape[0]
    H = W = int(HW ** 0.5)
    h2, w2 = H // 2, W // 2
    NT = H // TH
    w1 = jnp.transpose(iw.reshape(C, Cin)).astype(BF16)
    b1 = ib.reshape(1, C)
    w3 = jnp.transpose(lw, (2, 3, 1, 0)).reshape(3, 3 * C, C).astype(BF16)
    b3 = lb.reshape(1, C)

    TH2 = TH // 2
    in_specs = [
        pl.BlockSpec((1, TH * W, Cin), lambda n, t: (n, t, 0)),
        pl.BlockSpec((1, W, Cin), lambda n, t: (n, jnp.maximum(t * TH - 1, 0), 0)),
        pl.BlockSpec((1, W, Cin), lambda n, t: (n, jnp.minimum(t * TH + TH, H - 1), 0)),
        pl.BlockSpec((Cin, C), lambda n, t: (0, 0)),
        pl.BlockSpec((1, C), lambda n, t: (0, 0)),
        pl.BlockSpec((1, TH2 * w2, C), lambda n, t: (n, t, 0)),
        pl.BlockSpec((1, w2, C), lambda n, t: (n, jnp.maximum(t * TH2 - 1, 0), 0)),
        pl.BlockSpec((1, w2, C), lambda n, t: (n, jnp.minimum(t * TH2 + TH2, h2 - 1), 0)),
        pl.BlockSpec((3, 3 * C, C), lambda n, t: (0, 0, 0)),
        pl.BlockSpec((1, C), lambda n, t: (0, 0)),
    ]
    outs = pl.pallas_call(
        _make_l1_kernel(TH, W, C, NT),
        out_shape=[
            jax.ShapeDtypeStruct((N, HW, C), F32),
            jax.ShapeDtypeStruct((N, HW, C), BF16),
        ],
        grid=(N, NT),
        in_specs=in_specs,
        out_specs=[
            pl.BlockSpec((1, TH * W, C), lambda n, t: (n, t, 0)),
            pl.BlockSpec((1, TH * W, C), lambda n, t: (n, t, 0)),
        ],
        scratch_shapes=[
            pltpu.VMEM(((TH + 2) * W, C), BF16),
            pltpu.VMEM(((TH + 2) * W, C), BF16),
            pltpu.VMEM(((TH + 2) * W, C), BF16),
            pltpu.VMEM((TH * W, C), F32),
        ],
        compiler_params=pltpu.CompilerParams(
            dimension_semantics=("parallel", "arbitrary"),
            vmem_limit_bytes=100 * 1024 * 1024,
        ),
    )(x_hwc, x_hwc, x_hwc, w1, b1, td, td, td, w3, b3)
    return outs


def _make_l0_kernel(TH, W, C, NT):
    def kfn(xm_ref, xt_ref, xb_ref, w1_ref, b1_ref,
            tdm_ref, tdt_ref, tdb_ref, w3_ref, b3_ref,
            r_ref, bc_ref, bl_ref, br_ref, acc_ref):
        t = pl.program_id(1)
        w2 = W // 2
        cin = xm_ref.shape[1]

        def lat_dot(x2d):
            return lax.dot_general(x2d.astype(BF16), w1_ref[...],
                                   (((0,), (0,)), ((), ())),
                                   preferred_element_type=F32) + b1_ref[...]

        lat = lat_dot(xm_ref[0].reshape(cin, TH * W))
        lat = lat + _up2x_rows(tdm_ref[0].astype(F32), TH // 2, w2, C)
        main = lat.astype(BF16)

        top = lat_dot(xt_ref[0, :, 7, :])
        top = top + _up2x_cols(tdt_ref[0].astype(F32), w2, C)
        top = jnp.where(t > 0, top, 0.0).astype(BF16)

        bot = lat_dot(xb_ref[0, :, 0, :])
        bot = bot + _up2x_cols(tdb_ref[0].astype(F32), w2, C)
        bot = jnp.where(t < NT - 1, bot, 0.0).astype(BF16)

        val = jnp.concatenate([top, main, bot], axis=0)
        _conv3x3_acc(val, w3_ref, bc_ref, bl_ref, br_ref, acc_ref, TH, W, C)
        r = jnp.transpose(acc_ref[...]) + b3_ref[...]
        r_ref[0] = r.reshape(C, TH, W)
    return kfn


def _level0(x_nchw, iw, ib, lw, lb, td, TH=32):
    N, Cin, H, W = x_nchw.shape
    C = iw.shape[0]
    h2, w2 = H // 2, W // 2
    NT = H // TH
    w1 = jnp.transpose(iw.reshape(C, Cin)).astype(BF16)
    b1 = ib.reshape(1, C)
    w3 = jnp.transpose(lw, (2, 3, 1, 0)).reshape(3, 3 * C, C).astype(BF16)
    b3 = lb.reshape(C, 1)

    TH2 = TH // 2
    in_specs = [
        pl.BlockSpec((1, Cin, TH, W), lambda n, t: (n, 0, t, 0)),
        pl.BlockSpec((1, Cin, 8, W),
                     lambda n, t: (n, 0, jnp.maximum(t * (TH // 8) - 1, 0), 0)),
        pl.BlockSpec((1, Cin, 8, W),
                     lambda n, t: (n, 0, jnp.minimum((t + 1) * (TH // 8), H // 8 - 1), 0)),
        pl.BlockSpec((Cin, C), lambda n, t: (0, 0)),
        pl.BlockSpec((1, C), lambda n, t: (0, 0)),
        pl.BlockSpec((1, TH2 * w2, C), lambda n, t: (n, t, 0)),
        pl.BlockSpec((1, w2, C), lambda n, t: (n, jnp.maximum(t * TH2 - 1, 0), 0)),
        pl.BlockSpec((1, w2, C), lambda n, t: (n, jnp.minimum(t * TH2 + TH2, h2 - 1), 0)),
        pl.BlockSpec((3, 3 * C, C), lambda n, t: (0, 0, 0)),
        pl.BlockSpec((C, 1), lambda n, t: (0, 0)),
    ]
    out = pl.pallas_call(
        _make_l0_kernel(TH, W, C, NT),
        out_shape=jax.ShapeDtypeStruct((N, C, H, W), F32),
        grid=(N, NT),
        in_specs=in_specs,
        out_specs=pl.BlockSpec((1, C, TH, W), lambda n, t: (n, 0, t, 0)),
        scratch_shapes=[
            pltpu.VMEM(((TH + 2) * W, C), BF16),
            pltpu.VMEM(((TH + 2) * W, C), BF16),
            pltpu.VMEM(((TH + 2) * W, C), BF16),
            pltpu.VMEM((TH * W, C), F32),
        ],
        compiler_params=pltpu.CompilerParams(
            dimension_semantics=("parallel", "arbitrary"),
            vmem_limit_bytes=100 * 1024 * 1024,
        ),
    )(x_nchw, x_nchw, x_nchw, w1, b1, td, td, td, w3, b3)
    return out


def _to_nchw(r_hwc, N, C, H, W):
    return jnp.transpose(r_hwc, (0, 2, 1)).reshape(N, C, H, W)


def kernel(x0, x1, x2, iw0, ib0, lw0, lb0, iw1, ib1, lw1, lb1, iw2, ib2, lw2, lb2):
    N = x0.shape[0]
    C = iw0.shape[0]
    H0, H1, H2 = x0.shape[2], x1.shape[2], x2.shape[2]

    xh1 = jnp.transpose(x1, (0, 2, 3, 1)).reshape(N, H1 * H1, x1.shape[1])
    xh2 = jnp.transpose(x2, (0, 2, 3, 1)).reshape(N, H2 * H2, x2.shape[1])

    r2f, inner2, poolf = _level_whole(xh2, iw2, ib2, lw2, lb2, pool=True)

    r1f, inner1 = _level1(xh1, iw1, ib1, lw1, lb1, td=inner2)

    r0f = _level0(x0, iw0, ib0, lw0, lb0, td=inner1)

    r0 = r0f
    r1 = _to_nchw(r1f, N, C, H1, H1)
    r2 = _to_nchw(r2f, N, C, H2, H2)
    pool = _to_nchw(poolf, N, C, H2 // 2, H2 // 2)
    return (r0, r1, r2, pool)


# speedup vs baseline: 2.5969x; 1.0263x over previous
import jax
import jax.numpy as jnp
from jax import lax
from jax.experimental import pallas as pl
from jax.experimental.pallas import tpu as pltpu

BF16 = jnp.bfloat16
F32 = jnp.float32


def _up2x_rows(td, rows, w2, c):
    t = td.reshape(rows, w2, c)
    t = jnp.broadcast_to(t[:, :, None, :], (rows, w2, 2, c)).reshape(rows, 2 * w2, c)
    t = jnp.broadcast_to(t[:, None, :, :], (rows, 2, 2 * w2, c)).reshape(2 * rows, 2 * w2, c)
    return t.reshape(4 * rows * w2, c)


def _up2x_cols(row, w2, c):
    return jnp.broadcast_to(row[:, None, :], (w2, 2, c)).reshape(2 * w2, c)


def _conv3x3_acc(val, w3_ref, b3_row, bc_ref, bl_ref, br_ref, acc_ref, th, w, c):
    m2 = (th + 2) * w
    bc_ref[...] = val
    xix = lax.broadcasted_iota(jnp.int32, (m2, 1), 0) % w
    zrow = jnp.zeros((1, c), BF16)
    vl = jnp.concatenate([val[1:], zrow], axis=0)
    bl_ref[...] = jnp.where(xix == w - 1, zrow, vl)
    vr = jnp.concatenate([zrow, val[:-1]], axis=0)
    br_ref[...] = jnp.where(xix == 0, zrow, vr)
    acc_ref[...] = jnp.broadcast_to(b3_row, (th * w, c))
    for dy in range(3):
        s = pl.ds(dy * w, th * w)
        lhs = jnp.concatenate([br_ref[s], bc_ref[s], bl_ref[s]], axis=1)
        acc_ref[...] += jnp.dot(lhs, w3_ref[dy], preferred_element_type=F32)


def _whole_body(H, W, C, x_ref, w1_ref, b1_ref, td_ref, w3_ref, b3_ref,
                r_ref, inner_ref, pool_ref, bc_ref, bl_ref, br_ref, acc_ref):
    lat = lax.dot_general(x_ref[0].astype(BF16), w1_ref[...].astype(BF16),
                          (((1,), (1,)), ((), ())),
                          preferred_element_type=F32) + b1_ref[...]
    if td_ref is not None:
        lat = lat + _up2x_rows(td_ref[0].astype(F32), H // 2, W // 2, C)
    inner = lat.astype(BF16)
    inner_ref[0] = inner

    zr = jnp.zeros((W, C), BF16)
    val = jnp.concatenate([zr, inner, zr], axis=0)

    _conv3x3_acc(val, w3_ref, b3_ref[...], bc_ref, bl_ref, br_ref, acc_ref,
                 H, W, C)
    r_ref[0] = acc_ref[...]
    if pool_ref is not None:
        p = acc_ref[...].reshape(H // 2, 2, W // 2, 2, C)[:, 0, :, 0, :]
        pool_ref[0] = p.reshape((H // 2) * (W // 2), C)


def _level_whole(x_hwc, iw, ib, lw, lb, td=None, pool=False):
    N, HW, Cin = x_hwc.shape
    C = iw.shape[0]
    H = W = int(HW ** 0.5)
    assert H * W == HW
    w1 = iw.reshape(C, Cin)
    b1 = ib.reshape(1, C)
    w3 = jnp.transpose(lw, (2, 3, 1, 0)).reshape(3, 3 * C, C).astype(BF16)
    b3 = lb.reshape(1, C)

    in_specs = [
        pl.BlockSpec((1, HW, Cin), lambda n: (n, 0, 0)),
        pl.BlockSpec((C, Cin), lambda n: (0, 0)),
        pl.BlockSpec((1, C), lambda n: (0, 0)),
    ]
    args = [x_hwc, w1, b1]
    if td is not None:
        in_specs.append(pl.BlockSpec((1, HW // 4, C), lambda n: (n, 0, 0)))
        args.append(td)
    in_specs += [
        pl.BlockSpec((3, 3 * C, C), lambda n: (0, 0, 0)),
        pl.BlockSpec((1, C), lambda n: (0, 0)),
    ]
    args += [w3, b3]

    out_shape = [
        jax.ShapeDtypeStruct((N, HW, C), F32),
        jax.ShapeDtypeStruct((N, HW, C), BF16),
    ]
    out_specs = [
        pl.BlockSpec((1, HW, C), lambda n: (n, 0, 0)),
        pl.BlockSpec((1, HW, C), lambda n: (n, 0, 0)),
    ]
    if pool:
        out_shape.append(jax.ShapeDtypeStruct((N, HW // 4, C), F32))
        out_specs.append(pl.BlockSpec((1, HW // 4, C), lambda n: (n, 0, 0)))

    def kfn(*refs):
        it = iter(refs)
        x_ref = next(it); w1_ref = next(it); b1_ref = next(it)
        td_ref = next(it) if td is not None else None
        w3_ref = next(it); b3_ref = next(it)
        r_ref = next(it); inner_ref = next(it)
        pool_ref = next(it) if pool else None
        bc_ref = next(it); bl_ref = next(it); br_ref = next(it)
        acc_ref = next(it)
        _whole_body(H, W, C, x_ref, w1_ref, b1_ref, td_ref, w3_ref, b3_ref,
                    r_ref, inner_ref, pool_ref, bc_ref, bl_ref, br_ref, acc_ref)

    outs = pl.pallas_call(
        kfn,
        out_shape=out_shape,
        grid=(N,),
        in_specs=in_specs,
        out_specs=out_specs,
        scratch_shapes=[
            pltpu.VMEM(((H + 2) * W, C), BF16),
            pltpu.VMEM(((H + 2) * W, C), BF16),
            pltpu.VMEM(((H + 2) * W, C), BF16),
            pltpu.VMEM((HW, C), F32),
        ],
        compiler_params=pltpu.CompilerParams(
            dimension_semantics=("parallel",),
            vmem_limit_bytes=100 * 1024 * 1024,
        ),
    )(*args)
    return outs


def _make_l1_kernel(TH, W, C, NT):
    def kfn(xm_ref, xt_ref, xb_ref, w1_ref, b1_ref,
            tdm_ref, tdt_ref, tdb_ref, w3_ref, b3_ref,
            r_ref, inner_ref, bc_ref, bl_ref, br_ref, acc_ref):
        t = pl.program_id(1)
        w2 = W // 2

        def lat_dot(x2d):
            return lax.dot_general(x2d.astype(BF16), w1_ref[...].astype(BF16),
                                   (((1,), (1,)), ((), ())),
                                   preferred_element_type=F32) + b1_ref[...]

        lat = lat_dot(xm_ref[0])
        lat = lat + _up2x_rows(tdm_ref[0].astype(F32), TH // 2, w2, C)
        main = lat.astype(BF16)
        inner_ref[0] = main

        top = lat_dot(xt_ref[0])
        top = top + _up2x_cols(tdt_ref[0].astype(F32), w2, C)
        top = jnp.where(t > 0, top, 0.0).astype(BF16)

        bot = lat_dot(xb_ref[0])
        bot = bot + _up2x_cols(tdb_ref[0].astype(F32), w2, C)
        bot = jnp.where(t < NT - 1, bot, 0.0).astype(BF16)

        val = jnp.concatenate([top, main, bot], axis=0)
        _conv3x3_acc(val, w3_ref, b3_ref[...], bc_ref, bl_ref, br_ref, acc_ref,
                     TH, W, C)
        r_ref[0] = acc_ref[...]
    return kfn


def _level1(x_hwc, iw, ib, lw, lb, td, TH=32):
    N, HW, Cin = x_hwc.shape
    C = iw.shape[0]
    H = W = int(HW ** 0.5)
    h2, w2 = H // 2, W // 2
    NT = H // TH
    w1 = iw.reshape(C, Cin)
    b1 = ib.reshape(1, C)
    w3 = jnp.transpose(lw, (2, 3, 1, 0)).reshape(3, 3 * C, C).astype(BF16)
    b3 = lb.reshape(1, C)

    TH2 = TH // 2
    in_specs = [
        pl.BlockSpec((1, TH * W, Cin), lambda n, t: (n, t, 0)),
        pl.BlockSpec((1, W, Cin), lambda n, t: (n, jnp.maximum(t * TH - 1, 0), 0)),
        pl.BlockSpec((1, W, Cin), lambda n, t: (n, jnp.minimum(t * TH + TH, H - 1), 0)),
        pl.BlockSpec((C, Cin), lambda n, t: (0, 0)),
        pl.BlockSpec((1, C), lambda n, t: (0, 0)),
        pl.BlockSpec((1, TH2 * w2, C), lambda n, t: (n, t, 0)),
        pl.BlockSpec((1, w2, C), lambda n, t: (n, jnp.maximum(t * TH2 - 1, 0), 0)),
        pl.BlockSpec((1, w2, C), lambda n, t: (n, jnp.minimum(t * TH2 + TH2, h2 - 1), 0)),
        pl.BlockSpec((3, 3 * C, C), lambda n, t: (0, 0, 0)),
        pl.BlockSpec((1, C), lambda n, t: (0, 0)),
    ]
    outs = pl.pallas_call(
        _make_l1_kernel(TH, W, C, NT),
        out_shape=[
            jax.ShapeDtypeStruct((N, HW, C), F32),
            jax.ShapeDtypeStruct((N, HW, C), BF16),
        ],
        grid=(N, NT),
        in_specs=in_specs,
        out_specs=[
            pl.BlockSpec((1, TH * W, C), lambda n, t: (n, t, 0)),
            pl.BlockSpec((1, TH * W, C), lambda n, t: (n, t, 0)),
        ],
        scratch_shapes=[
            pltpu.VMEM(((TH + 2) * W, C), BF16),
            pltpu.VMEM(((TH + 2) * W, C), BF16),
            pltpu.VMEM(((TH + 2) * W, C), BF16),
            pltpu.VMEM((TH * W, C), F32),
        ],
        compiler_params=pltpu.CompilerParams(
            dimension_semantics=("parallel", "arbitrary"),
            vmem_limit_bytes=100 * 1024 * 1024,
        ),
    )(x_hwc, x_hwc, x_hwc, w1, b1, td, td, td, w3, b3)
    return outs


def _make_l0_kernel(TH, W, C, NT):
    def kfn(xm_ref, xt_ref, xb_ref, w1_ref, b1_ref,
            tdm_ref, tdt_ref, tdb_ref, w3_ref, b3_ref,
            r_ref, bc_ref, bl_ref, br_ref, acc_ref):
        t = pl.program_id(1)
        w2 = W // 2
        cin = xm_ref.shape[1]

        def lat_dot(x2d):
            return lax.dot_general(x2d.astype(BF16), w1_ref[...].astype(BF16),
                                   (((0,), (1,)), ((), ())),
                                   preferred_element_type=F32) + b1_ref[...]

        lat = lat_dot(xm_ref[0].reshape(cin, TH * W))
        lat = lat + _up2x_rows(tdm_ref[0].astype(F32), TH // 2, w2, C)
        main = lat.astype(BF16)

        top = lat_dot(xt_ref[0, :, 7, :])
        top = top + _up2x_cols(tdt_ref[0].astype(F32), w2, C)
        top = jnp.where(t > 0, top, 0.0).astype(BF16)

        bot = lat_dot(xb_ref[0, :, 0, :])
        bot = bot + _up2x_cols(tdb_ref[0].astype(F32), w2, C)
        bot = jnp.where(t < NT - 1, bot, 0.0).astype(BF16)

        val = jnp.concatenate([top, main, bot], axis=0)
        _conv3x3_acc(val, w3_ref, b3_ref[...], bc_ref, bl_ref, br_ref, acc_ref,
                     TH, W, C)
        r = jnp.transpose(acc_ref[...])
        r_ref[0] = r.reshape(C, TH, W)
    return kfn


def _level0(x_nchw, iw, ib, lw, lb, td, TH=32):
    N, Cin, H, W = x_nchw.shape
    C = iw.shape[0]
    h2, w2 = H // 2, W // 2
    NT = H // TH
    w1 = iw.reshape(C, Cin)
    b1 = ib.reshape(1, C)
    w3 = jnp.transpose(lw, (2, 3, 1, 0)).reshape(3, 3 * C, C).astype(BF16)
    b3 = lb.reshape(1, C)

    TH2 = TH // 2
    in_specs = [
        pl.BlockSpec((1, Cin, TH, W), lambda n, t: (n, 0, t, 0)),
        pl.BlockSpec((1, Cin, 8, W),
                     lambda n, t: (n, 0, jnp.maximum(t * (TH // 8) - 1, 0), 0)),
        pl.BlockSpec((1, Cin, 8, W),
                     lambda n, t: (n, 0, jnp.minimum((t + 1) * (TH // 8), H // 8 - 1), 0)),
        pl.BlockSpec((C, Cin), lambda n, t: (0, 0)),
        pl.BlockSpec((1, C), lambda n, t: (0, 0)),
        pl.BlockSpec((1, TH2 * w2, C), lambda n, t: (n, t, 0)),
        pl.BlockSpec((1, w2, C), lambda n, t: (n, jnp.maximum(t * TH2 - 1, 0), 0)),
        pl.BlockSpec((1, w2, C), lambda n, t: (n, jnp.minimum(t * TH2 + TH2, h2 - 1), 0)),
        pl.BlockSpec((3, 3 * C, C), lambda n, t: (0, 0, 0)),
        pl.BlockSpec((1, C), lambda n, t: (0, 0)),
    ]
    out = pl.pallas_call(
        _make_l0_kernel(TH, W, C, NT),
        out_shape=jax.ShapeDtypeStruct((N, C, H, W), F32),
        grid=(N, NT),
        in_specs=in_specs,
        out_specs=pl.BlockSpec((1, C, TH, W), lambda n, t: (n, 0, t, 0)),
        scratch_shapes=[
            pltpu.VMEM(((TH + 2) * W, C), BF16),
            pltpu.VMEM(((TH + 2) * W, C), BF16),
            pltpu.VMEM(((TH + 2) * W, C), BF16),
            pltpu.VMEM((TH * W, C), F32),
        ],
        compiler_params=pltpu.CompilerParams(
            dimension_semantics=("parallel", "arbitrary"),
            vmem_limit_bytes=100 * 1024 * 1024,
        ),
    )(x_nchw, x_nchw, x_nchw, w1, b1, td, td, td, w3, b3)
    return out


def _to_nchw(r_hwc, N, C, H, W):
    return jnp.transpose(r_hwc, (0, 2, 1)).reshape(N, C, H, W)


def kernel(x0, x1, x2, iw0, ib0, lw0, lb0, iw1, ib1, lw1, lb1, iw2, ib2, lw2, lb2):
    N = x0.shape[0]
    C = iw0.shape[0]
    H0, H1, H2 = x0.shape[2], x1.shape[2], x2.shape[2]

    xh1 = jnp.transpose(x1, (0, 2, 3, 1)).reshape(N, H1 * H1, x1.shape[1])
    xh2 = jnp.transpose(x2, (0, 2, 3, 1)).reshape(N, H2 * H2, x2.shape[1])

    r2f, inner2, poolf = _level_whole(xh2, iw2, ib2, lw2, lb2, pool=True)

    r1f, inner1 = _level1(xh1, iw1, ib1, lw1, lb1, td=inner2)

    r0f = _level0(x0, iw0, ib0, lw0, lb0, td=inner1)

    r0 = r0f
    r1 = _to_nchw(r1f, N, C, H1, H1)
    r2 = _to_nchw(r2f, N, C, H2, H2)
    pool = _to_nchw(poolf, N, C, H2 // 2, H2 // 2)
    return (r0, r1, r2, pool)
